# baseline (speedup 1.0000x reference)
import functools

import jax
import jax.numpy as jnp
from jax import lax
from jax.experimental import pallas as pl
from jax.experimental.pallas import tpu as pltpu
from jax.experimental.pallas import tpu_sc as plsc

B = 32
P = 1024
D = 64
K = 20
LP = 8
QB = 4
BIG = 3.0e38
IBIG = 1 << 30
KP = 24
NT2 = 2 * B


def _make_dist_body(width, center):
    def body(x1_ref, x2_ref, rv_ref, ri_ref, cv_ref, ci_ref):
        def col_reduce(mat):
            bits = lax.bitcast_convert_type(mat, jnp.int32)
            iio = lax.broadcasted_iota(jnp.int32, (P, P), 0)
            packed = lax.bitcast_convert_type((bits & ~1023) | iio,
                                              jnp.float32)
            pmin = jnp.min(packed, axis=0, keepdims=True)
            cidx = lax.bitcast_convert_type(pmin, jnp.int32) & 1023
            cmin = jnp.min(mat, axis=0, keepdims=True)
            return jnp.maximum(cmin, 0.0), cidx

        def reduce_full(a, b):
            ones = jnp.ones((P, 1), jnp.float32)
            a2 = jnp.sum(a * a, axis=1, keepdims=True)
            b2 = jnp.sum(b * b, axis=1, keepdims=True)
            af = jnp.concatenate([-2.0 * a, a2, ones], axis=1)
            bf = jnp.concatenate([b, ones, b2], axis=1)
            d2 = lax.dot_general(af, bf, (((1,), (1,)), ((), ())),
                                 preferred_element_type=jnp.float32)
            cminv, cidxv = col_reduce(d2)
            d2t = lax.dot_general(bf, af, (((1,), (1,)), ((), ())),
                                  preferred_element_type=jnp.float32)
            rminv, ridxv = col_reduce(d2t)
            return rminv, ridxv, cminv, cidxv

        if center:
            off = jnp.where(
                lax.broadcasted_iota(jnp.int32, (P, width), 1) < 2, 16.0, 0.0)
        for q in range(QB):
            a = x1_ref[q] - off if center else x1_ref[q]
            b = x2_ref[q] - off if center else x2_ref[q]
            rm, ri, cm, ci = reduce_full(a, b)
            rv_ref[q] = rm
            ri_ref[q] = ri
            cv_ref[q] = cm
            ci_ref[q] = ci

    return body


def _nn_reduce(m1, m2, width, center):
    out_v = jax.ShapeDtypeStruct((B, 1, P), jnp.float32)
    out_i = jax.ShapeDtypeStruct((B, 1, P), jnp.int32)
    spec = pl.BlockSpec((QB, 1, P), lambda b: (b, 0, 0))
    return pl.pallas_call(
        _make_dist_body(width, center),
        grid=(B // QB,),
        in_specs=[
            pl.BlockSpec((QB, P, width), lambda b: (b, 0, 0)),
            pl.BlockSpec((QB, P, width), lambda b: (b, 0, 0)),
        ],
        out_specs=[spec, spec, spec, spec],
        out_shape=[out_v, out_i, out_v, out_i],
    )(m1, m2)


def _sc_topk_gather(rv, ri, cv, ci, x1f, x2f):
    mesh = plsc.VectorSubcoreMesh(core_axis_name="c", subcore_axis_name="s")

    @functools.partial(
        pl.kernel,
        out_type=[jax.ShapeDtypeStruct((NT2 * KP, D), jnp.float32),
                  jax.ShapeDtypeStruct((NT2 * KP, D), jnp.float32)],
        mesh=mesh,
        compiler_params=pltpu.CompilerParams(needs_layout_passes=False,
                                             use_tc_tiling_on_sc=False),
        scratch_types=[
            pltpu.VMEM((2, 64, 16), jnp.float32),
            pltpu.VMEM((2, 64, 16), jnp.int32),
            pltpu.VMEM((2, 32), jnp.int32),
            pltpu.VMEM((2, 32), jnp.int32),
            pltpu.VMEM((2, 32, D), jnp.float32),
            pltpu.VMEM((2, 32, D), jnp.float32),
            pltpu.SemaphoreType.DMA,
            pltpu.SemaphoreType.DMA,
            pltpu.SemaphoreType.DMA,
        ],
    )
    def topk_kernel(rv_hbm, ri_hbm, cv_hbm, ci_hbm,
                    x1_hbm, x2_hbm, fi_hbm, fc_hbm,
                    vals_v, idx_v, gfi_v, gfc_v, rfi_v, rfc_v,
                    sem_in, sem_g, sem_out):
        wid = lax.axis_index("s") * 2 + lax.axis_index("c")
        lane = lax.iota(jnp.int32, 16)
        v_hbms = (rv_hbm, cv_hbm)
        i_hbms = (ri_hbm, ci_hbm)
        tabs = ((x1_hbm, x2_hbm), (x2_hbm, x1_hbm))

        in_h = []
        for k in range(2):
            in_h.append(pltpu.async_copy(v_hbms[k].at[wid], vals_v.at[k],
                                         sem_in))
            in_h.append(pltpu.async_copy(i_hbms[k].at[wid], idx_v.at[k],
                                         sem_in))

        g_h = []
        for k in range(2):
            in_h[2 * k].wait()
            in_h[2 * k + 1].wait()
            vk = vals_v.at[k]
            ik = idx_v.at[k]

            def build_step(j, carry, vk=vk):
                cm0, cm1, cm2, cm3 = carry
                s = jnp.min(vk[j])
                hit = lane == (j % 16)
                g = j // 16
                cm0 = jnp.where(hit & (g == 0), s, cm0)
                cm1 = jnp.where(hit & (g == 1), s, cm1)
                cm2 = jnp.where(hit & (g == 2), s, cm2)
                cm3 = jnp.where(hit & (g == 3), s, cm3)
                return cm0, cm1, cm2, cm3

            big = jnp.full((16,), BIG, jnp.float32)
            cms = lax.fori_loop(0, 64, build_step, (big, big, big, big),
                                unroll=4)

            def select_step(t, carry, vk=vk, ik=ik):
                fi0, fi1, fc0, fc1, cm0, cm1, cm2, cm3 = carry
                mval = jnp.min(jnp.minimum(jnp.minimum(cm0, cm1),
                                           jnp.minimum(cm2, cm3)))
                c0 = jnp.where(cm0 == mval, lane, IBIG)
                c1 = jnp.where(cm1 == mval, lane + 16, IBIG)
                c2 = jnp.where(cm2 == mval, lane + 32, IBIG)
                c3 = jnp.where(cm3 == mval, lane + 48, IBIG)
                jrow = jnp.min(jnp.minimum(jnp.minimum(c0, c1),
                                           jnp.minimum(c2, c3)))
                row = vk[jrow]
                lpos = plsc.all_reduce_ffs(row == mval)
                newrow = jnp.where(lane == lpos, BIG, row)
                vk[jrow] = newrow
                nm = jnp.min(newrow)
                hit = lane == (jrow % 16)
                g = jrow // 16
                cm0 = jnp.where(hit & (g == 0), nm, cm0)
                cm1 = jnp.where(hit & (g == 1), nm, cm1)
                cm2 = jnp.where(hit & (g == 2), nm, cm2)
                cm3 = jnp.where(hit & (g == 3), nm, cm3)
                jsplat = jnp.full((16,), jrow, jnp.int32)
                cand = plsc.load_gather(ik, [jsplat, lpos])
                pos = jrow * 16 + lpos
                fi_g = wid * P + pos
                fc_g = wid * P + cand
                sel0 = (lane == t) & (t < 16)
                sel1 = lane == (t - 16)
                fi0 = jnp.where(sel0, fi_g, fi0)
                fi1 = jnp.where(sel1, fi_g, fi1)
                fc0 = jnp.where(sel0, fc_g, fc0)
                fc1 = jnp.where(sel1, fc_g, fc1)
                return fi0, fi1, fc0, fc1, cm0, cm1, cm2, cm3

            z = jnp.zeros((16,), jnp.int32)
            fi0, fi1, fc0, fc1, _, _, _, _ = lax.fori_loop(
                0, K, select_step, (z, z, z, z) + cms)
            gfik = gfi_v.at[k]
            gfck = gfc_v.at[k]
            gfik[pl.ds(0, 16)] = fi0
            gfik[pl.ds(16, 16)] = fi1
            gfck[pl.ds(0, 16)] = fc0
            gfck[pl.ds(16, 16)] = fc1
            tin, tcand = tabs[k]
            g_h.append(pltpu.async_copy(tin.at[gfik], rfi_v.at[k], sem_g))
            g_h.append(pltpu.async_copy(tcand.at[gfck], rfc_v.at[k], sem_g))

        out_h = []
        for k in range(2):
            g_h[2 * k].wait()
            g_h[2 * k + 1].wait()
            rb = (k * B + wid) * KP
            out_h.append(pltpu.async_copy(rfi_v.at[k].at[pl.ds(0, KP)],
                                          fi_hbm.at[pl.ds(rb, KP)], sem_out))
            out_h.append(pltpu.async_copy(rfc_v.at[k].at[pl.ds(0, KP)],
                                          fc_hbm.at[pl.ds(rb, KP)], sem_out))
        for h in out_h:
            h.wait()

    return topk_kernel(rv, ri, cv, ci, x1f, x2f)


def _loss_body(fif_ref, fcf_ref, fil_ref, fcl_ref, g1_ref, g2_ref, o_ref):
    NR = B * KP
    rio = lax.broadcasted_iota(jnp.int32, (NR, 1), 0)
    mask = jnp.where(rio % KP < K, 1.0, 0.0)
    n = B * K

    def vicreg(x, y, msk, n):
        inv = jnp.sum(msk * (x - y) ** 2) / (n * D)

        def vc(z):
            mu = jnp.sum(msk * z, axis=0, keepdims=True) * (1.0 / n)
            zc = msk * (z - mu)
            var = jnp.sum(zc * zc, axis=0) * (1.0 / n)
            std = jnp.sqrt(var + 1e-4)
            v = jnp.sum(jnp.maximum(1.0 - std, 0.0)) / D
            cov = lax.dot_general(zc, zc, (((0,), (0,)), ((), ())),
                                  preferred_element_type=jnp.float32)
            cov = cov * (1.0 / (n - 1))
            eye = (lax.broadcasted_iota(jnp.int32, (D, D), 0)
                   == lax.broadcasted_iota(jnp.int32, (D, D), 1))
            off = jnp.where(eye, 0.0, cov)
            c = jnp.sum(off * off) / D
            return v, c

        vx, cx = vc(x)
        vy, cy = vc(y)
        return inv, vx + vy, cx + cy

    ones = jnp.ones((B, 1), jnp.float32)
    g_inv, g_var, g_cov = vicreg(g1_ref[...], g2_ref[...], ones, B)
    l_inv = jnp.float32(0.0)
    l_var = jnp.float32(0.0)
    l_cov = jnp.float32(0.0)
    for fr, cr in ((fif_ref, fcf_ref), (fil_ref, fcl_ref)):
        for d in range(2):
            i, v, cv = vicreg(fr[d], cr[d], mask, n)
            l_inv += i
            l_var += v
            l_cov += cv
    o_ref[0] = g_inv
    o_ref[1] = g_var
    o_ref[2] = g_cov
    o_ref[3] = l_inv * 0.25
    o_ref[4] = l_var * 0.25
    o_ref[5] = l_cov * 0.25


def _losses(fif, fcf, fil, fcl, x1_glob, x2_glob):
    return pl.pallas_call(
        _loss_body,
        out_specs=pl.BlockSpec(memory_space=pltpu.SMEM),
        out_shape=jax.ShapeDtypeStruct((6,), jnp.float32),
    )(fif, fcf, fil, fcl, x1_glob, x2_glob)


def kernel(x1_maps, x2_maps, x1_glob, x2_glob, x1_locations, x2_locations):
    l1p = jnp.pad(x1_locations, ((0, 0), (0, 0), (0, LP - 2)))
    l2p = jnp.pad(x2_locations, ((0, 0), (0, 0), (0, LP - 2)))
    x1f = x1_maps.reshape(B * P, D)
    x2f = x2_maps.reshape(B * P, D)
    shp = (B, 64, 16)

    frv, fri, fcv, fci = _nn_reduce(x1_maps, x2_maps, D, False)
    fif, fcf = _sc_topk_gather(frv.reshape(shp), fri.reshape(shp),
                               fcv.reshape(shp), fci.reshape(shp), x1f, x2f)
    lrv, lri, lcv, lci = _nn_reduce(l1p, l2p, LP, True)
    fil, fcl = _sc_topk_gather(lrv.reshape(shp), lri.reshape(shp),
                               lcv.reshape(shp), lci.reshape(shp), x1f, x2f)

    return _losses(fif.reshape(2, B * KP, D), fcf.reshape(2, B * KP, D),
                   fil.reshape(2, B * KP, D), fcl.reshape(2, B * KP, D),
                   x1_glob, x2_glob)
